# Initial kernel scaffold; baseline (speedup 1.0000x reference)
import jax
import jax.numpy as jnp
from jax.experimental import pallas as pl


def kernel(program_x, program_noise, program_edge_index, program_node_cluster, program_node_ratio, voxel_x, voxel_noise, voxel_edge_index, cross_edge_index, enc_W, enc_b, msg_W0, msg_b0, upd_W0, upd_b0, msg_W1, msg_b1, upd_W1, upd_b1, venc_W, venc_b, att_W1, att_b1, att_W2, att_b2, vmsg_W, vmsg_b, vupd_W, vupd_b):
    raise NotImplementedError("write your pallas kernel here")


# speedup vs baseline: 2.6606x; 2.6606x over previous
import functools

import jax
import jax.numpy as jnp
from jax import lax
from jax.experimental import pallas as pl
from jax.experimental.pallas import tpu as pltpu
from jax.experimental.pallas import tpu_sc as plsc

H = 128
NCLUSTER = 100
NCP = 104
NB = 128
NTILES = 32
ZR = 64
NP8 = 10240
NV8 = 51200


def _fused(xs, ws, b, act, res, BN):
    N = xs[0].shape[0]
    Hout = ws[0].shape[1]
    nx = len(xs)
    in_specs = [pl.BlockSpec((BN, x.shape[1]), lambda i: (i, 0)) for x in xs]
    in_specs += [pl.BlockSpec(w.shape, lambda i: (0, 0)) for w in ws]
    args = list(xs) + list(ws)
    if b is not None:
        in_specs.append(pl.BlockSpec((1, Hout), lambda i: (0, 0)))
        args.append(b.reshape(1, Hout))
    if res is not None:
        in_specs.append(pl.BlockSpec((BN, Hout), lambda i: (i, 0)))
        args.append(res)

    def body(*refs):
        out = refs[-1]
        acc = jnp.dot(refs[0][...], refs[nx][...],
                      preferred_element_type=jnp.float32)
        for i in range(1, nx):
            acc = acc + jnp.dot(refs[i][...], refs[nx + i][...],
                                preferred_element_type=jnp.float32)
        k = 2 * nx
        if b is not None:
            acc = acc + refs[k][...]
            k += 1
        if act:
            acc = jnp.maximum(acc, 0.01 * acc)
        if res is not None:
            acc = refs[k][...] + acc
        out[...] = acc

    return pl.pallas_call(
        body,
        grid=(N // BN,),
        in_specs=in_specs,
        out_specs=pl.BlockSpec((BN, Hout), lambda i: (i, 0)),
        out_shape=jax.ShapeDtypeStruct((N, Hout), jnp.float32),
    )(*args)


def _cluster_stats(x, cluster, n_valid, BN):
    N = x.shape[0]

    def body(x_ref, c_ref, s_ref, n_ref):
        i = pl.program_id(0)

        @pl.when(i == 0)
        def _():
            s_ref[...] = jnp.zeros_like(s_ref)
            n_ref[...] = jnp.zeros_like(n_ref)

        cl = c_ref[...]
        row = lax.broadcasted_iota(jnp.int32, (BN, 1), 0) + i * BN
        valid = row < n_valid
        cols = lax.broadcasted_iota(jnp.int32, (BN, NCP), 1)
        oh = jnp.where((cols == cl) & valid, 1.0, 0.0)
        dn = (((0,), (0,)), ((), ()))
        s_ref[...] += lax.dot_general(oh, x_ref[...], dn,
                                      preferred_element_type=jnp.float32)
        n_ref[...] += lax.dot_general(oh, jnp.ones((BN, H), jnp.float32),
                                      dn, preferred_element_type=jnp.float32)

    return pl.pallas_call(
        body,
        grid=(N // BN,),
        in_specs=[pl.BlockSpec((BN, H), lambda i: (i, 0)),
                  pl.BlockSpec((BN, 1), lambda i: (i, 0))],
        out_specs=[pl.BlockSpec((NCP, H), lambda i: (0, 0)),
                   pl.BlockSpec((NCP, H), lambda i: (0, 0))],
        out_shape=[jax.ShapeDtypeStruct((NCP, H), jnp.float32),
                   jax.ShapeDtypeStruct((NCP, H), jnp.float32)],
    )(x, cluster)


def _prog_update(x, a0, a1, c0, c1, S, Cc, cluster, ratio, uW, ub, BN):
    N = x.shape[0]

    def body(x_ref, a0_ref, a1_ref, c0_ref, c1_ref, s_ref, cc_ref,
             cl_ref, r_ref, w_ref, b_ref, out_ref):
        xb = x_ref[...]
        aggr = (a0_ref[...] + a1_ref[...]) * (
            1.0 / jnp.maximum(c0_ref[...] + c1_ref[...], 1.0))
        tab = s_ref[...] / jnp.maximum(cc_ref[...], 1.0)
        cl = cl_ref[...]
        cols = lax.broadcasted_iota(jnp.int32, (BN, NCP), 1)
        oh = jnp.where(cols == cl, 1.0, 0.0)
        rsum = jnp.sum(r_ref[...], axis=1, keepdims=True)
        c = jnp.dot(oh, tab, preferred_element_type=jnp.float32) * rsum
        acc = jnp.dot(xb, w_ref[0:H, :], preferred_element_type=jnp.float32)
        acc += jnp.dot(aggr, w_ref[H:2 * H, :],
                       preferred_element_type=jnp.float32)
        acc += jnp.dot(c, w_ref[2 * H:3 * H, :],
                       preferred_element_type=jnp.float32)
        acc += b_ref[...]
        out_ref[...] = xb + jnp.maximum(acc, 0.01 * acc)

    return pl.pallas_call(
        body,
        grid=(N // BN,),
        in_specs=[pl.BlockSpec((BN, H), lambda i: (i, 0)),
                  pl.BlockSpec((BN, H), lambda i: (i, 0)),
                  pl.BlockSpec((BN, H), lambda i: (i, 0)),
                  pl.BlockSpec((BN, 1), lambda i: (i, 0)),
                  pl.BlockSpec((BN, 1), lambda i: (i, 0)),
                  pl.BlockSpec((NCP, H), lambda i: (0, 0)),
                  pl.BlockSpec((NCP, H), lambda i: (0, 0)),
                  pl.BlockSpec((BN, 1), lambda i: (i, 0)),
                  pl.BlockSpec((BN, 10), lambda i: (i, 0)),
                  pl.BlockSpec((3 * H, H), lambda i: (0, 0)),
                  pl.BlockSpec((1, H), lambda i: (0, 0))],
        out_specs=pl.BlockSpec((BN, H), lambda i: (i, 0)),
        out_shape=jax.ShapeDtypeStruct((N, H), jnp.float32),
    )(x, a0, a1, c0, c1, S, Cc, cluster, ratio, uW, ub.reshape(1, H))


def _tanh_dot(z, w2, BN):
    E = z.shape[0]

    def body(z_ref, w_ref, out_ref):
        t = jnp.tanh(z_ref[...])
        out_ref[...] = jnp.sum(t * w_ref[...], axis=1, keepdims=True)

    return pl.pallas_call(
        body,
        grid=(E // BN,),
        in_specs=[pl.BlockSpec((BN, H), lambda i: (i, 0)),
                  pl.BlockSpec((1, H), lambda i: (0, 0))],
        out_specs=pl.BlockSpec((BN, 1), lambda i: (i, 0)),
        out_shape=jax.ShapeDtypeStruct((E, 1), jnp.float32),
    )(z, w2.reshape(1, H))


def _softmax_stats(att, BN):
    E = att.shape[0]

    def body(a_ref, out_ref, m_sc, s_sc):
        i = pl.program_id(0)

        @pl.when(i == 0)
        def _():
            m_sc[0] = -jnp.inf
            s_sc[0] = 0.0

        blk = a_ref[...]
        bm = jnp.max(blk)
        m_old = m_sc[0]
        m_new = jnp.maximum(m_old, bm)
        s_sc[0] = s_sc[0] * jnp.exp(m_old - m_new) + jnp.sum(
            jnp.exp(blk - m_new))
        m_sc[0] = m_new

        @pl.when(i == pl.num_programs(0) - 1)
        def _():
            col = lax.broadcasted_iota(jnp.int32, (1, 2), 1)
            out_ref[...] = jnp.where(col == 0, m_sc[0], s_sc[0])

    return pl.pallas_call(
        body,
        grid=(E // BN,),
        in_specs=[pl.BlockSpec((BN, 1), lambda i: (i, 0))],
        out_specs=pl.BlockSpec((1, 2), lambda i: (0, 0)),
        out_shape=jax.ShapeDtypeStruct((1, 2), jnp.float32),
        scratch_shapes=[pltpu.SMEM((1,), jnp.float32),
                        pltpu.SMEM((1,), jnp.float32)],
    )(att)


def _aw_kernel(att, stats, BN):
    E = att.shape[0]

    def body(a_ref, st_ref, out_ref):
        m = st_ref[0, 0]
        s = st_ref[0, 1]
        out_ref[...] = jnp.exp(a_ref[...] - m) / s

    return pl.pallas_call(
        body,
        grid=(E // BN,),
        in_specs=[pl.BlockSpec((BN, 1), lambda i: (i, 0)),
                  pl.BlockSpec((1, 2), lambda i: (0, 0))],
        out_specs=pl.BlockSpec((BN, 1), lambda i: (i, 0)),
        out_shape=jax.ShapeDtypeStruct((E, 1), jnp.float32),
    )(att, stats)


def _add_cross(vf, a0, a1, BN):
    N = vf.shape[0]
    nacc_blocks = a0.shape[0] // BN

    def amap(i):
        return (jnp.minimum(i, nacc_blocks - 1), 0)

    def body(v_ref, a0_ref, a1_ref, out_ref):
        i = pl.program_id(0)
        add = jnp.where(i < nacc_blocks, a0_ref[...] + a1_ref[...], 0.0)
        out_ref[...] = v_ref[...] + add

    return pl.pallas_call(
        body,
        grid=(N // BN,),
        in_specs=[pl.BlockSpec((BN, H), lambda i: (i, 0)),
                  pl.BlockSpec((BN, H), amap),
                  pl.BlockSpec((BN, H), amap)],
        out_specs=pl.BlockSpec((BN, H), lambda i: (i, 0)),
        out_shape=jax.ShapeDtypeStruct((N, H), jnp.float32),
    )(vf, a0, a1)


def _voxel_update(vf, chunks0, chunks1, c0, c1, uW, ub, BN):
    N = vf.shape[0]
    nch = len(chunks0)

    def body(*refs):
        v_ref = refs[0]
        a0 = [refs[1 + i] for i in range(nch)]
        a1 = [refs[1 + nch + i] for i in range(nch)]
        c0_ref = refs[1 + 2 * nch]
        c1_ref = refs[2 + 2 * nch]
        w_ref = refs[3 + 2 * nch]
        b_ref = refs[4 + 2 * nch]
        out_ref = refs[-1]
        vb = v_ref[...]
        va = jnp.concatenate([r[...] for r in a0], axis=1)
        va = va + jnp.concatenate([r[...] for r in a1], axis=1)
        mean = va * (1.0 / jnp.maximum(c0_ref[...] + c1_ref[...], 1.0))
        acc = jnp.dot(vb, w_ref[0:H, :], preferred_element_type=jnp.float32)
        acc += jnp.dot(mean, w_ref[H:2 * H, :],
                       preferred_element_type=jnp.float32)
        acc += b_ref[...]
        out_ref[...] = vb + jnp.maximum(acc, 0.01 * acc)

    D = chunks0[0].shape[1]
    in_specs = [pl.BlockSpec((BN, H), lambda i: (i, 0))]
    in_specs += [pl.BlockSpec((BN, D), lambda i: (i, 0))] * (2 * nch)
    in_specs += [pl.BlockSpec((BN, 1), lambda i: (i, 0))] * 2
    in_specs += [pl.BlockSpec((2 * H, H), lambda i: (0, 0)),
                 pl.BlockSpec((1, H), lambda i: (0, 0))]
    return pl.pallas_call(
        body,
        grid=(N // BN,),
        in_specs=in_specs,
        out_specs=pl.BlockSpec((BN, H), lambda i: (i, 0)),
        out_shape=jax.ShapeDtypeStruct((N, H), jnp.float32),
    )(vf, *chunks0, *chunks1, c0, c1, uW, ub.reshape(1, H))


def _sc_mesh():
    return plsc.VectorSubcoreMesh(core_axis_name="c", subcore_axis_name="s",
                                  num_cores=2, num_subcores=16)


def _fill_zero2d(ref, rows, width):
    zero = jnp.zeros((16,), jnp.float32)

    def body(r, _):
        for c in range(width // 16):
            ref[r, pl.ds(c * 16, 16)] = zero
        return 0

    lax.fori_loop(0, rows, body, 0)


def _fill_const1d(ref, n, val):
    v = jnp.full((16,), val, jnp.float32)

    def body(r, _):
        ref[pl.ds(r * 16, 16)] = v
        return 0

    lax.fori_loop(0, n // 16, body, 0)


def _sc_seg_sum(A, B, dst, src, n_acc):
    D = A.shape[1]
    E = dst.shape[0]
    nb = E // (NTILES * NB)
    zper = n_acc // 16

    out_type = [jax.ShapeDtypeStruct((2, n_acc, D), jnp.float32),
                jax.ShapeDtypeStruct((2, n_acc), jnp.float32)]
    scratch = [
        pltpu.VMEM((NB,), jnp.int32),
        pltpu.VMEM((NB,), jnp.int32),
        pltpu.VMEM((NB, D), jnp.float32),
        pltpu.VMEM((NB, D), jnp.float32),
        pltpu.VMEM((NB,), jnp.float32),
        pltpu.VMEM((ZR, D), jnp.float32),
        pltpu.VMEM((ZR,), jnp.float32),
        pltpu.VMEM_SHARED((n_acc, D), jnp.float32),
        pltpu.VMEM_SHARED((n_acc,), jnp.float32),
        pltpu.SemaphoreType.DMA,
        pltpu.SemaphoreType.DMA,
    ]

    @functools.partial(pl.kernel, out_type=out_type, mesh=_sc_mesh(),
                       scratch_types=scratch)
    def k(A_h, B_h, dst_h, src_h, acc_o, cnt_o, dstv, srcv, rowsA, rowsB,
          ones, zbuf, zcnt, acc_sh, cnt_sh, s1, s2):
        cid = lax.axis_index("c")
        sid = lax.axis_index("s")
        wid = cid * 16 + sid
        _fill_zero2d(zbuf, ZR, D)
        _fill_const1d(zcnt, ZR, 0.0)
        _fill_const1d(ones, NB, 1.0)

        def zloop(j, _):
            off = sid * zper + j * ZR
            pltpu.sync_copy(zbuf, acc_sh.at[pl.ds(off, ZR)])
            pltpu.sync_copy(zcnt, cnt_sh.at[pl.ds(off, ZR)])
            return 0

        lax.fori_loop(0, zper // ZR, zloop, 0)
        plsc.subcore_barrier()

        e0 = wid * (nb * NB)

        def eloop(j, _):
            base = e0 + j * NB
            pltpu.sync_copy(dst_h.at[pl.ds(base, NB)], dstv)
            pltpu.sync_copy(src_h.at[pl.ds(base, NB)], srcv)
            ca = pltpu.async_copy(A_h.at[dstv], rowsA, s1)
            cb = pltpu.async_copy(B_h.at[srcv], rowsB, s2)
            ca.wait()
            cb.wait()

            def vbody(kk, _):
                for c in range(D // 16):
                    sl = pl.ds(c * 16, 16)
                    v = rowsA[kk, sl] + rowsB[kk, sl]
                    rowsA[kk, sl] = jnp.maximum(v, 0.01 * v)
                return 0

            lax.fori_loop(0, NB, vbody, 0)
            pltpu.sync_copy(rowsA, acc_sh.at[dstv], add=True)
            pltpu.sync_copy(ones, cnt_sh.at[dstv], add=True)
            return 0

        lax.fori_loop(0, nb, eloop, 0)
        plsc.subcore_barrier()

        @pl.when(sid == 0)
        def _():
            pltpu.sync_copy(acc_sh, acc_o.at[cid])
            pltpu.sync_copy(cnt_sh, cnt_o.at[cid])

    return k(A, B, dst, src)


def _sc_gather_add(U, V, pe, ve):
    D = U.shape[1]
    E = pe.shape[0]
    nb = E // (NTILES * NB)
    scratch = [
        pltpu.VMEM((NB,), jnp.int32),
        pltpu.VMEM((NB,), jnp.int32),
        pltpu.VMEM((NB, D), jnp.float32),
        pltpu.VMEM((NB, D), jnp.float32),
        pltpu.SemaphoreType.DMA,
        pltpu.SemaphoreType.DMA,
    ]

    @functools.partial(pl.kernel,
                       out_type=jax.ShapeDtypeStruct((E, D), jnp.float32),
                       mesh=_sc_mesh(), scratch_types=scratch)
    def k(U_h, V_h, pe_h, ve_h, z_o, pev, vev, rowsU, rowsV, s1, s2):
        cid = lax.axis_index("c")
        sid = lax.axis_index("s")
        wid = cid * 16 + sid
        e0 = wid * (nb * NB)

        def eloop(j, _):
            base = e0 + j * NB
            pltpu.sync_copy(pe_h.at[pl.ds(base, NB)], pev)
            pltpu.sync_copy(ve_h.at[pl.ds(base, NB)], vev)
            ca = pltpu.async_copy(U_h.at[pev], rowsU, s1)
            cb = pltpu.async_copy(V_h.at[vev], rowsV, s2)
            ca.wait()
            cb.wait()

            def vbody(kk, _):
                for c in range(D // 16):
                    sl = pl.ds(c * 16, 16)
                    rowsU[kk, sl] = rowsU[kk, sl] + rowsV[kk, sl]
                return 0

            lax.fori_loop(0, NB, vbody, 0)
            pltpu.sync_copy(rowsU, z_o.at[pl.ds(base, NB)])
            return 0

        lax.fori_loop(0, nb, eloop, 0)

    return k(U, V, pe, ve)


def _sc_weighted_scatter(X, pe, ve, aw, n_acc):
    D = X.shape[1]
    E = pe.shape[0]
    nb = E // (NTILES * NB)
    zper = n_acc // 16
    scratch = [
        pltpu.VMEM((NB,), jnp.int32),
        pltpu.VMEM((NB,), jnp.int32),
        pltpu.VMEM((NB,), jnp.float32),
        pltpu.VMEM((NB, D), jnp.float32),
        pltpu.VMEM((ZR, D), jnp.float32),
        pltpu.VMEM_SHARED((n_acc, D), jnp.float32),
        pltpu.SemaphoreType.DMA,
    ]

    @functools.partial(pl.kernel,
                       out_type=jax.ShapeDtypeStruct((2, n_acc, D),
                                                     jnp.float32),
                       mesh=_sc_mesh(), scratch_types=scratch)
    def k(X_h, pe_h, ve_h, aw_h, acc_o, pev, vev, awv, rows, zbuf,
          acc_sh, s1):
        cid = lax.axis_index("c")
        sid = lax.axis_index("s")
        wid = cid * 16 + sid
        _fill_zero2d(zbuf, ZR, D)

        def zloop(j, _):
            pltpu.sync_copy(zbuf, acc_sh.at[pl.ds(sid * zper + j * ZR, ZR)])
            return 0

        lax.fori_loop(0, zper // ZR, zloop, 0)
        plsc.subcore_barrier()
        e0 = wid * (nb * NB)

        def eloop(j, _):
            base = e0 + j * NB
            pltpu.sync_copy(pe_h.at[pl.ds(base, NB)], pev)
            pltpu.sync_copy(ve_h.at[pl.ds(base, NB)], vev)
            pltpu.sync_copy(aw_h.at[pl.ds(base, NB)], awv)
            pltpu.async_copy(X_h.at[pev], rows, s1).wait()

            def gbody(g, _):
                t16 = awv[pl.ds(g * 16, 16)]
                for j in range(16):
                    s = lax.gather(
                        t16, jnp.full((16, 1), j, jnp.int32),
                        lax.GatherDimensionNumbers(
                            offset_dims=(), collapsed_slice_dims=(0,),
                            start_index_map=(0,)),
                        (1,),
                        mode=lax.GatherScatterMode.PROMISE_IN_BOUNDS)
                    kk = g * 16 + j
                    for c in range(D // 16):
                        sl = pl.ds(c * 16, 16)
                        rows[kk, sl] = rows[kk, sl] * s
                return 0

            lax.fori_loop(0, NB // 16, gbody, 0)
            pltpu.sync_copy(rows, acc_sh.at[vev], add=True)
            return 0

        lax.fori_loop(0, nb, eloop, 0)
        plsc.subcore_barrier()

        @pl.when(sid == 0)
        def _():
            pltpu.sync_copy(acc_sh, acc_o.at[cid])

    return k(X, pe, ve, aw)


def _sc_seg_sum_chunked(Astk, Bstk, dst, src, n_tab, n_acc, n_chunks):
    D = Astk.shape[1]
    E = dst.shape[0]
    nb = E // (NTILES * NB)
    zper = n_acc // 16
    out_type = [
        jax.ShapeDtypeStruct((2, n_chunks, n_acc, D), jnp.float32),
        jax.ShapeDtypeStruct((2, n_acc), jnp.float32),
    ]
    scratch = [
        pltpu.VMEM((NB,), jnp.int32),
        pltpu.VMEM((NB,), jnp.int32),
        pltpu.VMEM((NB,), jnp.int32),
        pltpu.VMEM((NB,), jnp.int32),
        pltpu.VMEM((NB, D), jnp.float32),
        pltpu.VMEM((NB, D), jnp.float32),
        pltpu.VMEM((NB,), jnp.float32),
        pltpu.VMEM((ZR, D), jnp.float32),
        pltpu.VMEM((ZR,), jnp.float32),
        pltpu.VMEM_SHARED((n_acc, D), jnp.float32),
        pltpu.VMEM_SHARED((n_acc,), jnp.float32),
        pltpu.SemaphoreType.DMA,
        pltpu.SemaphoreType.DMA,
    ]

    @functools.partial(pl.kernel, out_type=out_type, mesh=_sc_mesh(),
                       scratch_types=scratch,
                       compiler_params=pltpu.CompilerParams(
                           use_tc_tiling_on_sc=False))
    def k(A_h, B_h, dst_h, src_h, acc_o, cnt_o, dstv, srcv, ia, ib, rowsA,
          rowsB, ones, zbuf, zcnt, acc_sh, cnt_sh, s1, s2):
        cid = lax.axis_index("c")
        sid = lax.axis_index("s")
        wid = cid * 16 + sid
        _fill_zero2d(zbuf, ZR, D)
        _fill_const1d(zcnt, ZR, 0.0)
        _fill_const1d(ones, NB, 1.0)
        e0 = wid * (nb * NB)

        for ch in range(n_chunks):
            def zloop(j, _):
                off = sid * zper + j * ZR
                pltpu.sync_copy(zbuf, acc_sh.at[pl.ds(off, ZR)])
                if ch == 0:
                    pltpu.sync_copy(zcnt, cnt_sh.at[pl.ds(off, ZR)])
                return 0

            lax.fori_loop(0, zper // ZR, zloop, 0)
            plsc.subcore_barrier()

            def eloop(j, _):
                base = e0 + j * NB
                pltpu.sync_copy(dst_h.at[pl.ds(base, NB)], dstv)
                pltpu.sync_copy(src_h.at[pl.ds(base, NB)], srcv)

                def oloop(i, _):
                    sl = pl.ds(i * 16, 16)
                    ia[sl] = dstv[sl] + ch * n_tab
                    ib[sl] = srcv[sl] + ch * n_tab
                    return 0

                lax.fori_loop(0, NB // 16, oloop, 0)
                ca = pltpu.async_copy(A_h.at[ia], rowsA, s1)
                cb = pltpu.async_copy(B_h.at[ib], rowsB, s2)
                ca.wait()
                cb.wait()

                def vbody(kk, _):
                    for c in range(D // 16):
                        sl = pl.ds(c * 16, 16)
                        v = rowsA[kk, sl] + rowsB[kk, sl]
                        rowsA[kk, sl] = jnp.maximum(v, 0.01 * v)
                    return 0

                lax.fori_loop(0, NB, vbody, 0)
                pltpu.sync_copy(rowsA, acc_sh.at[dstv], add=True)
                if ch == 0:
                    pltpu.sync_copy(ones, cnt_sh.at[dstv], add=True)
                return 0

            lax.fori_loop(0, nb, eloop, 0)
            plsc.subcore_barrier()

            @pl.when(sid == 0)
            def _():
                pltpu.sync_copy(acc_sh, acc_o.at[cid, ch])
                if ch == 0:
                    pltpu.sync_copy(cnt_sh, cnt_o.at[cid])

            plsc.subcore_barrier()

    return k(Astk, Bstk, dst, src)


def _pad_rows(a, n):
    return jnp.pad(a, ((0, n - a.shape[0]),) + ((0, 0),) * (a.ndim - 1))


def _pad_edges(e, junk, total):
    return jnp.concatenate(
        [e, jnp.full((total - e.shape[0],), junk, e.dtype)])


def kernel(program_x, program_noise, program_edge_index, program_node_cluster,
           program_node_ratio, voxel_x, voxel_noise, voxel_edge_index,
           cross_edge_index, enc_W, enc_b, msg_W0, msg_b0, upd_W0, upd_b0,
           msg_W1, msg_b1, upd_W1, upd_b1, venc_W, venc_b, att_W1, att_b1,
           att_W2, att_b2, vmsg_W, vmsg_b, vupd_W, vupd_b):
    NP = program_x.shape[0]
    NV = voxel_x.shape[0]
    EP = program_edge_index.shape[1]
    EC = cross_edge_index.shape[1]
    EV = voxel_edge_index.shape[1]
    EP8 = -(-EP // (NTILES * NB)) * (NTILES * NB)
    EC8 = -(-EC // (NTILES * NB)) * (NTILES * NB)
    EV8 = -(-EV // (NTILES * NB)) * (NTILES * NB)

    px = _pad_rows(program_x, NP8)
    pn = _pad_rows(program_noise, NP8)
    cl2 = _pad_rows(program_node_cluster.reshape(NP, 1), NP8)
    ratio = _pad_rows(program_node_ratio, NP8)
    vx = _pad_rows(voxel_x, NV8)
    vn = _pad_rows(voxel_noise, NV8)

    dstp = _pad_edges(program_edge_index[1], NP, EP8)
    srcp = _pad_edges(program_edge_index[0], NP, EP8)
    pep = _pad_edges(cross_edge_index[0], NP, EC8)
    vep = _pad_edges(cross_edge_index[1], NP, EC8)
    vdp = _pad_edges(voxel_edge_index[1], NV, EV8)
    vsp = _pad_edges(voxel_edge_index[0], NV, EV8)

    x = _fused([px, pn], [enc_W[:128], enc_W[128:]], enc_b, True, None, 1280)

    for (mW, mb, uW, ub) in ((msg_W0, msg_b0, upd_W0, upd_b0),
                             (msg_W1, msg_b1, upd_W1, upd_b1)):
        A = _fused([x], [mW[:H]], mb, False, None, 1280)
        B = _fused([x], [mW[H:]], None, False, None, 1280)
        accs, cnts = _sc_seg_sum(A, B, dstp, srcp, NP8)
        S, Cc = _cluster_stats(x, cl2, NP, 1280)
        x = _prog_update(x, accs[0], accs[1], cnts[0].reshape(NP8, 1),
                         cnts[1].reshape(NP8, 1), S, Cc, cl2, ratio, uW,
                         ub, 1280)

    vf = _fused([vx, vn], [venc_W[:16], venc_W[16:]], venc_b, True, None,
                1280)

    U = _fused([x], [att_W1[:H]], att_b1, False, None, 1280)
    Vt = _fused([vf[:NP8]], [att_W1[H:]], None, False, None, 1280)
    z = _sc_gather_add(U, Vt, pep, vep)
    att = _tanh_dot(z, att_W2[:, 0], 6272)
    stats = _softmax_stats(att[:EC], 4000)
    aw_full = _aw_kernel(att, stats, 6272)
    cacc = _sc_weighted_scatter(x, pep, vep, aw_full.reshape(EC8), NP8)
    vf = _add_cross(vf, cacc[0], cacc[1], 1280)

    pos = vx[:, :3]
    Av = _fused([vf, pos], [vmsg_W[:H], vmsg_W[2 * H:]], vmsg_b, False,
                None, 1280)
    Bv = _fused([vf, pos], [vmsg_W[H:2 * H], -vmsg_W[2 * H:]], None, False,
                None, 1280)
    Astk = Av.reshape(NV8, 4, 32).transpose(1, 0, 2).reshape(4 * NV8, 32)
    Bstk = Bv.reshape(NV8, 4, 32).transpose(1, 0, 2).reshape(4 * NV8, 32)
    vaccs, vcnts = _sc_seg_sum_chunked(Astk, Bstk, vdp, vsp, NV8, NV8, 4)
    vf = _voxel_update(vf, [vaccs[0, c] for c in range(4)],
                       [vaccs[1, c] for c in range(4)],
                       vcnts[0].reshape(NV8, 1), vcnts[1].reshape(NV8, 1),
                       vupd_W, vupd_b, 1280)
    return (vf[:NV], aw_full[:EC])
